# baseline (device time: 74775 ns/iter reference)
import functools

import jax
import jax.numpy as jnp
from jax import lax
from jax.experimental import pallas as pl
from jax.experimental.pallas import tpu as pltpu

N_DEV = 4
SKIP_DOTS = False

SEND_ORDER = (2, 1, 3)
RECV_ORDER = (1, 3, 2)


def kernel(x, w_mat):
    m_per, k = x.shape
    _, n = w_mat.shape
    n_per = n // N_DEV
    m_total = m_per * N_DEV

    n_half = n_per // 2
    n_chunks = 2 * N_DEV

    def body(x_hbm, w_hbm, out_hbm, x_ref, y_ref, w0, w1, w2, w3,
             amax_tx, amax_rx, q_tx, q_rx, stage, x_sem, w_sems, out_sems,
             a_send_sems, a_recv_sems, d_send_sems, d_recv_sems):
        my = lax.axis_index("i")

        w_bufs = [w0, w1, w2, w3]
        x_copy = pltpu.make_async_copy(x_hbm, x_ref, x_sem)
        x_copy.start()
        w_copies = []
        for h in range(n_chunks):
            w_copies.append(pltpu.make_async_copy(
                w_hbm.at[:, pl.ds(h * n_half, n_half)],
                w_bufs[h % 4],
                w_sems.at[h % 4],
            ))
        for h in range(4):
            w_copies[h].start()

        barrier_sem = pltpu.get_barrier_semaphore()
        for off in range(1, N_DEV):
            peer = lax.rem(my + off, N_DEV)
            pl.semaphore_signal(barrier_sem, inc=1, device_id=(peer,),
                                device_id_type=pl.DeviceIdType.MESH)
        pl.semaphore_wait(barrier_sem, N_DEV - 1)

        x_copy.wait()
        amaxes = []
        for j in range(N_DEV):
            for hh in range(2):
                h = 2 * j + hh
                w_copies[h].wait()
                if SKIP_DOTS:
                    blk = (x_ref[:, h * n_half:(h + 1) * n_half]
                           + w_bufs[h % 4][0, 0])
                else:
                    blk = lax.dot_general(
                        x_ref[:, :], w_bufs[h % 4][:, :],
                        (((1,), (0,)), ((), ())),
                        preferred_element_type=jnp.float32,
                    )
                if h + 4 < n_chunks:
                    w_copies[h + 4].start()
                y_ref[j, :, hh * n_half:(hh + 1) * n_half] = blk
                amaxes.append(jnp.max(jnp.abs(blk)))
        amax_local = functools.reduce(jnp.maximum, amaxes)

        amax_tx[:, :] = jnp.full((1, 128), amax_local, jnp.float32)
        amax_rx[pl.ds(my, 1), :] = jnp.full((1, 128), amax_local, jnp.float32)
        for off in range(1, N_DEV):
            peer = lax.rem(my + off, N_DEV)
            pltpu.make_async_remote_copy(
                src_ref=amax_tx,
                dst_ref=amax_rx.at[pl.ds(my, 1), :],
                send_sem=a_send_sems.at[off],
                recv_sem=a_recv_sems.at[my],
                device_id=(peer,),
                device_id_type=pl.DeviceIdType.MESH,
            ).start()
        for off in range(1, N_DEV):
            origin = lax.rem(my + off, N_DEV)
            pltpu.make_async_remote_copy(
                src_ref=amax_tx,
                dst_ref=amax_rx.at[pl.ds(origin, 1), :],
                send_sem=a_send_sems.at[0],
                recv_sem=a_recv_sems.at[origin],
                device_id=(origin,),
                device_id_type=pl.DeviceIdType.MESH,
            ).wait_recv()

        g_amax = jnp.max(amax_rx[:, :])
        scale = g_amax / 127.0
        inv_scale = 127.0 / g_amax

        def quantize(v):
            return jnp.clip(jnp.round(v * inv_scale), -127.0, 127.0)

        send_descs = []
        for off in SEND_ORDER:
            target = lax.rem(my + off, N_DEV)
            q_tx[off, :, :] = quantize(y_ref[target, :, :]).astype(jnp.int8)
            d = pltpu.make_async_remote_copy(
                src_ref=q_tx.at[off],
                dst_ref=q_rx.at[my],
                send_sem=d_send_sems.at[off],
                recv_sem=d_recv_sems.at[my],
                device_id=(target,),
                device_id_type=pl.DeviceIdType.MESH,
            )
            d.start()
            send_descs.append(d)

        stage[pl.ds(0, m_per), :] = quantize(y_ref[my, :, :]) * scale
        own_out = pltpu.make_async_copy(
            stage.at[pl.ds(0, m_per), :],
            out_hbm.at[pl.ds(my * m_per, m_per), :],
            out_sems.at[0],
        )
        own_out.start()

        out_copies = [own_out]
        for off in RECV_ORDER:
            origin = lax.rem(my + off, N_DEV)
            pltpu.make_async_remote_copy(
                src_ref=q_tx.at[0],
                dst_ref=q_rx.at[origin],
                send_sem=d_send_sems.at[0],
                recv_sem=d_recv_sems.at[origin],
                device_id=(origin,),
                device_id_type=pl.DeviceIdType.MESH,
            ).wait_recv()
            stage[pl.ds(off * m_per, m_per), :] = (
                q_rx[origin, :, :].astype(jnp.float32) * scale)
            oc = pltpu.make_async_copy(
                stage.at[pl.ds(off * m_per, m_per), :],
                out_hbm.at[pl.ds(origin * m_per, m_per), :],
                out_sems.at[off],
            )
            oc.start()
            out_copies.append(oc)

        for oc in out_copies:
            oc.wait()
        for d in send_descs:
            d.wait_send()
        for off in range(1, N_DEV):
            peer = lax.rem(my + off, N_DEV)
            pltpu.make_async_remote_copy(
                src_ref=amax_tx, dst_ref=amax_rx.at[pl.ds(my, 1), :],
                send_sem=a_send_sems.at[off], recv_sem=a_recv_sems.at[my],
                device_id=(peer,), device_id_type=pl.DeviceIdType.MESH,
            ).wait_send()

    return pl.pallas_call(
        body,
        out_shape=jax.ShapeDtypeStruct((m_total, n_per), jnp.float32),
        in_specs=[
            pl.BlockSpec(memory_space=pl.ANY),
            pl.BlockSpec(memory_space=pl.ANY),
        ],
        out_specs=pl.BlockSpec(memory_space=pl.ANY),
        scratch_shapes=[
            pltpu.VMEM((m_per, k), jnp.float32),
            pltpu.VMEM((N_DEV, m_per, n_per), jnp.float32),
            pltpu.VMEM((k, n_per // 2), jnp.float32),
            pltpu.VMEM((k, n_per // 2), jnp.float32),
            pltpu.VMEM((k, n_per // 2), jnp.float32),
            pltpu.VMEM((k, n_per // 2), jnp.float32),
            pltpu.VMEM((1, 128), jnp.float32),
            pltpu.VMEM((N_DEV, 128), jnp.float32),
            pltpu.VMEM((N_DEV, m_per, n_per), jnp.int8),
            pltpu.VMEM((N_DEV, m_per, n_per), jnp.int8),
            pltpu.VMEM((m_total, n_per), jnp.float32),
            pltpu.SemaphoreType.DMA,
            pltpu.SemaphoreType.DMA((4,)),
            pltpu.SemaphoreType.DMA((N_DEV,)),
            pltpu.SemaphoreType.DMA((N_DEV,)),
            pltpu.SemaphoreType.DMA((N_DEV,)),
            pltpu.SemaphoreType.DMA((N_DEV,)),
            pltpu.SemaphoreType.DMA((N_DEV,)),
        ],
        compiler_params=pltpu.CompilerParams(
            collective_id=0,
            vmem_limit_bytes=100 * 1024 * 1024,
        ),
    )(x, w_mat)


# device time: 54341 ns/iter; 1.3760x vs baseline; 1.3760x over previous
import functools

import jax
import jax.numpy as jnp
from jax import lax
from jax.experimental import pallas as pl
from jax.experimental.pallas import tpu as pltpu

N_DEV = 4
SKIP_DOTS = False

SEND_ORDER = (2, 1, 3)
RECV_ORDER = (1, 3, 2)


def kernel(x, w_mat):
    m_per, k = x.shape
    _, n = w_mat.shape
    n_per = n // N_DEV
    m_total = m_per * N_DEV

    m_half = m_per // 2

    def body(x_hbm, w_hbm, out_hbm, x_ref, y_ref, w_buf_a, w_buf_b,
             amax_tx, amax_rx, q_tx, q_rx, stage, x_sems, w_sems, out_sems,
             a_send_sems, a_recv_sems, d_send_sems, d_recv_sems):
        my = lax.axis_index("i")

        w_bufs = [w_buf_a, w_buf_b]
        x_copies = [
            pltpu.make_async_copy(
                x_hbm.at[pl.ds(hx * m_half, m_half), :],
                x_ref.at[pl.ds(hx * m_half, m_half), :],
                x_sems.at[hx],
            )
            for hx in range(2)
        ]
        w_copies = []
        for j in range(N_DEV):
            w_copies.append(pltpu.make_async_copy(
                w_hbm.at[:, pl.ds(j * n_per, n_per)],
                w_bufs[j % 2],
                w_sems.at[j % 2],
            ))
        x_copies[0].start()
        w_copies[0].start()
        x_copies[1].start()
        w_copies[1].start()

        barrier_sem = pltpu.get_barrier_semaphore()
        for off in range(1, N_DEV):
            peer = lax.rem(my + off, N_DEV)
            pl.semaphore_signal(barrier_sem, inc=1, device_id=(peer,),
                                device_id_type=pl.DeviceIdType.MESH)
        pl.semaphore_wait(barrier_sem, N_DEV - 1)

        amaxes = []
        for j in range(N_DEV):
            w_copies[j].wait()
            for hx in range(2):
                if j == 0:
                    x_copies[hx].wait()
                if SKIP_DOTS:
                    blk = (x_ref[pl.ds(hx * m_half, m_half),
                                 j * n_per:(j + 1) * n_per]
                           + w_bufs[j % 2][0, 0])
                else:
                    blk = lax.dot_general(
                        x_ref[hx * m_half:(hx + 1) * m_half, :],
                        w_bufs[j % 2][:, :],
                        (((1,), (0,)), ((), ())),
                        preferred_element_type=jnp.float32,
                    )
                if hx == 1 and j + 2 < N_DEV:
                    w_copies[j + 2].start()
                y_ref[j, hx * m_half:(hx + 1) * m_half, :] = blk
                amaxes.append(jnp.max(jnp.abs(blk)))
        amax_local = functools.reduce(jnp.maximum, amaxes)

        amax_tx[:, :] = jnp.full((1, 128), amax_local, jnp.float32)
        amax_rx[pl.ds(my, 1), :] = jnp.full((1, 128), amax_local, jnp.float32)
        for off in range(1, N_DEV):
            peer = lax.rem(my + off, N_DEV)
            pltpu.make_async_remote_copy(
                src_ref=amax_tx,
                dst_ref=amax_rx.at[pl.ds(my, 1), :],
                send_sem=a_send_sems.at[off],
                recv_sem=a_recv_sems.at[my],
                device_id=(peer,),
                device_id_type=pl.DeviceIdType.MESH,
            ).start()
        for off in range(1, N_DEV):
            origin = lax.rem(my + off, N_DEV)
            pltpu.make_async_remote_copy(
                src_ref=amax_tx,
                dst_ref=amax_rx.at[pl.ds(origin, 1), :],
                send_sem=a_send_sems.at[0],
                recv_sem=a_recv_sems.at[origin],
                device_id=(origin,),
                device_id_type=pl.DeviceIdType.MESH,
            ).wait_recv()

        g_amax = jnp.max(amax_rx[:, :])
        scale = g_amax / 127.0
        inv_scale = 127.0 / g_amax

        def quantize(v):
            return jnp.clip(jnp.round(v * inv_scale), -127.0, 127.0)

        send_descs = []
        for off in SEND_ORDER:
            target = lax.rem(my + off, N_DEV)
            q_tx[off, :, :] = quantize(y_ref[target, :, :]).astype(jnp.int8)
            d = pltpu.make_async_remote_copy(
                src_ref=q_tx.at[off],
                dst_ref=q_rx.at[my],
                send_sem=d_send_sems.at[off],
                recv_sem=d_recv_sems.at[my],
                device_id=(target,),
                device_id_type=pl.DeviceIdType.MESH,
            )
            d.start()
            send_descs.append(d)

        stage[pl.ds(0, m_per), :] = quantize(y_ref[my, :, :]) * scale
        own_out = pltpu.make_async_copy(
            stage.at[pl.ds(0, m_per), :],
            out_hbm.at[pl.ds(my * m_per, m_per), :],
            out_sems.at[0],
        )
        own_out.start()

        out_copies = [own_out]
        for off in RECV_ORDER:
            origin = lax.rem(my + off, N_DEV)
            pltpu.make_async_remote_copy(
                src_ref=q_tx.at[0],
                dst_ref=q_rx.at[origin],
                send_sem=d_send_sems.at[0],
                recv_sem=d_recv_sems.at[origin],
                device_id=(origin,),
                device_id_type=pl.DeviceIdType.MESH,
            ).wait_recv()
            stage[pl.ds(off * m_per, m_per), :] = (
                q_rx[origin, :, :].astype(jnp.float32) * scale)
            oc = pltpu.make_async_copy(
                stage.at[pl.ds(off * m_per, m_per), :],
                out_hbm.at[pl.ds(origin * m_per, m_per), :],
                out_sems.at[off],
            )
            oc.start()
            out_copies.append(oc)

        for oc in out_copies:
            oc.wait()
        for d in send_descs:
            d.wait_send()
        for off in range(1, N_DEV):
            peer = lax.rem(my + off, N_DEV)
            pltpu.make_async_remote_copy(
                src_ref=amax_tx, dst_ref=amax_rx.at[pl.ds(my, 1), :],
                send_sem=a_send_sems.at[off], recv_sem=a_recv_sems.at[my],
                device_id=(peer,), device_id_type=pl.DeviceIdType.MESH,
            ).wait_send()

    return pl.pallas_call(
        body,
        out_shape=jax.ShapeDtypeStruct((m_total, n_per), jnp.float32),
        in_specs=[
            pl.BlockSpec(memory_space=pl.ANY),
            pl.BlockSpec(memory_space=pl.ANY),
        ],
        out_specs=pl.BlockSpec(memory_space=pl.ANY),
        scratch_shapes=[
            pltpu.VMEM((m_per, k), jnp.float32),
            pltpu.VMEM((N_DEV, m_per, n_per), jnp.float32),
            pltpu.VMEM((k, n_per), jnp.float32),
            pltpu.VMEM((k, n_per), jnp.float32),
            pltpu.VMEM((1, 128), jnp.float32),
            pltpu.VMEM((N_DEV, 128), jnp.float32),
            pltpu.VMEM((N_DEV, m_per, n_per), jnp.int8),
            pltpu.VMEM((N_DEV, m_per, n_per), jnp.int8),
            pltpu.VMEM((m_total, n_per), jnp.float32),
            pltpu.SemaphoreType.DMA((2,)),
            pltpu.SemaphoreType.DMA((2,)),
            pltpu.SemaphoreType.DMA((N_DEV,)),
            pltpu.SemaphoreType.DMA((N_DEV,)),
            pltpu.SemaphoreType.DMA((N_DEV,)),
            pltpu.SemaphoreType.DMA((N_DEV,)),
            pltpu.SemaphoreType.DMA((N_DEV,)),
        ],
        compiler_params=pltpu.CompilerParams(
            collective_id=0,
            vmem_limit_bytes=100 * 1024 * 1024,
        ),
    )(x, w_mat)


# device time: 53758 ns/iter; 1.3910x vs baseline; 1.0108x over previous
import functools

import jax
import jax.numpy as jnp
from jax import lax
from jax.experimental import pallas as pl
from jax.experimental.pallas import tpu as pltpu

N_DEV = 4
SKIP_DOTS = False

SEND_ORDER = (2, 1, 3)
RECV_ORDER = (1, 3, 2)


def kernel(x, w_mat):
    m_per, k = x.shape
    _, n = w_mat.shape
    n_per = n // N_DEV
    m_total = m_per * N_DEV

    m_half = m_per // 2

    def body(x_hbm, w_hbm, out_hbm, x_ref, y_ref, w_buf_a, w_buf_b,
             amax_tx, amax_rx, q_tx, q_rx1, q_rx2, q_rx3, stage,
             x_sems, w_sems, out_sems,
             a_send_sems, a_recv_sems, d_send_sems, d_recv_sems):
        q_rx = [None, q_rx1, q_rx2, q_rx3]
        my = lax.axis_index("i")

        w_bufs = [w_buf_a, w_buf_b]
        x_copies = [
            pltpu.make_async_copy(
                x_hbm.at[pl.ds(hx * m_half, m_half), :],
                x_ref.at[pl.ds(hx * m_half, m_half), :],
                x_sems.at[hx],
            )
            for hx in range(2)
        ]
        w_copies = []
        for j in range(N_DEV):
            w_copies.append(pltpu.make_async_copy(
                w_hbm.at[:, pl.ds(j * n_per, n_per)],
                w_bufs[j % 2],
                w_sems.at[j % 2],
            ))
        x_copies[0].start()
        w_copies[0].start()
        x_copies[1].start()
        w_copies[1].start()

        barrier_sem = pltpu.get_barrier_semaphore()
        for off in range(1, N_DEV):
            peer = lax.rem(my + off, N_DEV)
            pl.semaphore_signal(barrier_sem, inc=1, device_id=(peer,),
                                device_id_type=pl.DeviceIdType.MESH)
        pl.semaphore_wait(barrier_sem, N_DEV - 1)

        amaxes = []
        for j in range(N_DEV):
            w_copies[j].wait()
            for hx in range(2):
                if j == 0:
                    x_copies[hx].wait()
                if SKIP_DOTS:
                    blk = (x_ref[pl.ds(hx * m_half, m_half),
                                 j * n_per:(j + 1) * n_per]
                           + w_bufs[j % 2][0, 0])
                else:
                    blk = lax.dot_general(
                        x_ref[hx * m_half:(hx + 1) * m_half, :],
                        w_bufs[j % 2][:, :],
                        (((1,), (0,)), ((), ())),
                        preferred_element_type=jnp.float32,
                    )
                if hx == 1 and j + 2 < N_DEV:
                    w_copies[j + 2].start()
                y_ref[j, hx * m_half:(hx + 1) * m_half, :] = blk
                amaxes.append(jnp.max(jnp.abs(blk)))
        amax_local = functools.reduce(jnp.maximum, amaxes)

        amax_tx[:, :] = jnp.full((1, 128), amax_local, jnp.float32)
        amax_rx[pl.ds(my, 1), :] = jnp.full((1, 128), amax_local, jnp.float32)
        for off in range(1, N_DEV):
            peer = lax.rem(my + off, N_DEV)
            pltpu.make_async_remote_copy(
                src_ref=amax_tx,
                dst_ref=amax_rx.at[pl.ds(my, 1), :],
                send_sem=a_send_sems.at[off],
                recv_sem=a_recv_sems.at[my],
                device_id=(peer,),
                device_id_type=pl.DeviceIdType.MESH,
            ).start()
        for off in range(1, N_DEV):
            origin = lax.rem(my + off, N_DEV)
            pltpu.make_async_remote_copy(
                src_ref=amax_tx,
                dst_ref=amax_rx.at[pl.ds(origin, 1), :],
                send_sem=a_send_sems.at[0],
                recv_sem=a_recv_sems.at[origin],
                device_id=(origin,),
                device_id_type=pl.DeviceIdType.MESH,
            ).wait_recv()

        g_amax = jnp.max(amax_rx[:, :])
        scale = g_amax / 127.0
        inv_scale = 127.0 / g_amax

        def quantize(v):
            return jnp.clip(jnp.round(v * inv_scale), -127.0, 127.0)

        send_descs = []
        for off in SEND_ORDER:
            target = lax.rem(my + off, N_DEV)
            q_tx[off, :, :] = quantize(y_ref[target, :, :]).astype(jnp.int8)
            d = pltpu.make_async_remote_copy(
                src_ref=q_tx.at[off],
                dst_ref=q_rx[N_DEV - off],
                send_sem=d_send_sems.at[off],
                recv_sem=d_recv_sems.at[N_DEV - off],
                device_id=(target,),
                device_id_type=pl.DeviceIdType.MESH,
            )
            d.start()
            send_descs.append(d)

        stage[pl.ds(0, m_per), :] = quantize(y_ref[my, :, :]) * scale
        own_out = pltpu.make_async_copy(
            stage.at[pl.ds(0, m_per), :],
            out_hbm.at[pl.ds(my * m_per, m_per), :],
            out_sems.at[0],
        )
        own_out.start()

        out_copies = [own_out]
        for s in (3, 1, 2):
            origin = lax.rem(my + N_DEV - s, N_DEV)
            pltpu.make_async_remote_copy(
                src_ref=q_tx.at[0],
                dst_ref=q_rx[s],
                send_sem=d_send_sems.at[0],
                recv_sem=d_recv_sems.at[s],
                device_id=(origin,),
                device_id_type=pl.DeviceIdType.MESH,
            ).wait_recv()
            stage[pl.ds(s * m_per, m_per), :] = (
                q_rx[s][:, :].astype(jnp.float32) * scale)
            oc = pltpu.make_async_copy(
                stage.at[pl.ds(s * m_per, m_per), :],
                out_hbm.at[pl.ds(origin * m_per, m_per), :],
                out_sems.at[s],
            )
            oc.start()
            out_copies.append(oc)

        for oc in out_copies:
            oc.wait()
        for d in send_descs:
            d.wait_send()
        for off in range(1, N_DEV):
            peer = lax.rem(my + off, N_DEV)
            pltpu.make_async_remote_copy(
                src_ref=amax_tx, dst_ref=amax_rx.at[pl.ds(my, 1), :],
                send_sem=a_send_sems.at[off], recv_sem=a_recv_sems.at[my],
                device_id=(peer,), device_id_type=pl.DeviceIdType.MESH,
            ).wait_send()

    return pl.pallas_call(
        body,
        out_shape=jax.ShapeDtypeStruct((m_total, n_per), jnp.float32),
        in_specs=[
            pl.BlockSpec(memory_space=pl.ANY),
            pl.BlockSpec(memory_space=pl.ANY),
        ],
        out_specs=pl.BlockSpec(memory_space=pl.ANY),
        scratch_shapes=[
            pltpu.VMEM((m_per, k), jnp.float32),
            pltpu.VMEM((N_DEV, m_per, n_per), jnp.float32),
            pltpu.VMEM((k, n_per), jnp.float32),
            pltpu.VMEM((k, n_per), jnp.float32),
            pltpu.VMEM((1, 128), jnp.float32),
            pltpu.VMEM((N_DEV, 128), jnp.float32),
            pltpu.VMEM((N_DEV, m_per, n_per), jnp.int8),
            pltpu.VMEM((m_per, n_per), jnp.int8),
            pltpu.VMEM((m_per, n_per), jnp.int8),
            pltpu.VMEM((m_per, n_per), jnp.int8),
            pltpu.VMEM((m_total, n_per), jnp.float32),
            pltpu.SemaphoreType.DMA((2,)),
            pltpu.SemaphoreType.DMA((2,)),
            pltpu.SemaphoreType.DMA((N_DEV,)),
            pltpu.SemaphoreType.DMA((N_DEV,)),
            pltpu.SemaphoreType.DMA((N_DEV,)),
            pltpu.SemaphoreType.DMA((N_DEV,)),
            pltpu.SemaphoreType.DMA((N_DEV,)),
        ],
        compiler_params=pltpu.CompilerParams(
            collective_id=0,
            vmem_limit_bytes=100 * 1024 * 1024,
        ),
    )(x, w_mat)
